# initial kernel scaffold (unmeasured)
import functools

import jax
import jax.numpy as jnp
from jax import lax
from jax.experimental import pallas as pl
from jax.experimental.pallas import tpu as pltpu

N_DEV = 16
E_PER = 4


def kernel(x, router_W, route_idx, expert_W, shared_W):
    m_per, d_model = x.shape
    e_per, _, d_ff = expert_W.shape
    n_exp = router_W.shape[1]

    def body(x_ref, rw_ref, idx_ref, ew_ref, sw_ref, out_ref,
             comm_ref, send_sems, recv_sems):
        my_pos = lax.axis_index("i")
        left = lax.rem(my_pos - 1 + N_DEV, N_DEV)
        right = lax.rem(my_pos + 1, N_DEV)

        barrier_sem = pltpu.get_barrier_semaphore()
        for nbr in [left, right]:
            pl.semaphore_signal(
                barrier_sem, inc=1,
                device_id=(nbr,), device_id_type=pl.DeviceIdType.MESH,
            )
        pl.semaphore_wait(barrier_sem, 2)

        x_f32 = x_ref[...]
        scores = jnp.dot(x_f32, rw_ref[...],
                         preferred_element_type=jnp.float32)
        s_max = jnp.max(scores, axis=-1, keepdims=True)
        p = jnp.exp(scores - s_max)
        p = p / jnp.sum(p, axis=-1, keepdims=True)
        idx = idx_ref[...]
        cols = lax.broadcasted_iota(jnp.int32, (m_per, n_exp), 1)
        psel = jnp.sum(jnp.where(cols == idx, p, 0.0), axis=-1,
                       keepdims=True)

        xb = x_f32.astype(jnp.bfloat16)
        out_ref[...] = jnp.dot(xb, sw_ref[...].astype(jnp.bfloat16),
                               preferred_element_type=jnp.float32)

        comm_ref[0] = ew_ref[...].astype(jnp.bfloat16)

        for h in range(N_DEV):
            slot = h % 2
            nxt = (h + 1) % 2
            if h < N_DEV - 1:
                rdma = pltpu.make_async_remote_copy(
                    src_ref=comm_ref.at[slot],
                    dst_ref=comm_ref.at[nxt],
                    send_sem=send_sems.at[slot],
                    recv_sem=recv_sems.at[nxt],
                    device_id=(right,),
                    device_id_type=pl.DeviceIdType.MESH,
                )
                rdma.start()

            owner = lax.rem(my_pos - h + N_DEV, N_DEV)
            acc = out_ref[...]
            for k in range(E_PER):
                e = owner * E_PER + k
                scale = jnp.where(idx == e, psel, 0.0)
                xm = (x_f32 * scale).astype(jnp.bfloat16)
                acc = acc + jnp.dot(xm, comm_ref[slot, k],
                                    preferred_element_type=jnp.float32)
            out_ref[...] = acc

            if h < N_DEV - 1:
                rdma.wait()

    return pl.pallas_call(
        body,
        out_shape=jax.ShapeDtypeStruct((m_per, d_ff), jnp.float32),
        in_specs=[pl.BlockSpec(memory_space=pltpu.VMEM)] * 5,
        out_specs=pl.BlockSpec(memory_space=pltpu.VMEM),
        scratch_shapes=[
            pltpu.VMEM((2, e_per, d_model, d_ff), jnp.bfloat16),
            pltpu.SemaphoreType.DMA((2,)),
            pltpu.SemaphoreType.DMA((2,)),
        ],
        compiler_params=pltpu.CompilerParams(collective_id=0),
    )(x, router_W, route_idx, expert_W, shared_W)


# baseline (device time: 732542 ns/iter reference)
import jax
import jax.numpy as jnp
from jax import lax
from jax.experimental import pallas as pl
from jax.experimental.pallas import tpu as pltpu

N_DEV = 16
E_PER = 4
TILE = 256


def kernel(x, router_W, route_idx, expert_W, shared_W):
    m_per, d_model = x.shape
    e_per, _, d_ff = expert_W.shape
    n_exp = router_W.shape[1]
    n_tiles = m_per // TILE

    def body(x_ref, rw_ref, idx_ref, ew_ref, sw_ref, out_ref,
             xs_ref, swb_ref, comm_ref, send_sems, recv_sems):
        my_pos = lax.axis_index("i")
        left = lax.rem(my_pos - 1 + N_DEV, N_DEV)
        right = lax.rem(my_pos + 1, N_DEV)

        barrier_sem = pltpu.get_barrier_semaphore()
        for nbr in [left, right]:
            pl.semaphore_signal(
                barrier_sem, inc=1,
                device_id=(nbr,), device_id_type=pl.DeviceIdType.MESH,
            )
        pl.semaphore_wait(barrier_sem, 2)

        for k in range(E_PER):
            comm_ref[0, k] = ew_ref[k].astype(jnp.bfloat16)
        swb_ref[...] = sw_ref[...].astype(jnp.bfloat16)

        def prep(t, _):
            sl = pl.ds(t * TILE, TILE)
            xt = x_ref[sl, :]
            scores = jnp.dot(xt, rw_ref[...],
                             preferred_element_type=jnp.float32)
            s_max = jnp.max(scores, axis=-1, keepdims=True)
            p = jnp.exp(scores - s_max)
            p = p / jnp.sum(p, axis=-1, keepdims=True)
            idxt = idx_ref[sl, :]
            cols = lax.broadcasted_iota(jnp.int32, (TILE, n_exp), 1)
            psel = jnp.sum(jnp.where(cols == idxt, p, 0.0), axis=-1,
                           keepdims=True)
            xs_ref[sl, :] = (xt * psel).astype(jnp.bfloat16)
            out_ref[sl, :] = jnp.dot(xt.astype(jnp.bfloat16), swb_ref[...],
                                     preferred_element_type=jnp.float32)
            return 0

        lax.fori_loop(0, n_tiles, prep, 0)

        for h in range(N_DEV):
            slot = h % 2
            nxt = (h + 1) % 2
            if h < N_DEV - 1:
                rdma = pltpu.make_async_remote_copy(
                    src_ref=comm_ref.at[slot],
                    dst_ref=comm_ref.at[nxt],
                    send_sem=send_sems.at[slot],
                    recv_sem=recv_sems.at[nxt],
                    device_id=(right,),
                    device_id_type=pl.DeviceIdType.MESH,
                )
                rdma.start()

            owner = lax.rem(my_pos - h + N_DEV, N_DEV)

            def tile_body(t, _, slot=slot, owner=owner):
                sl = pl.ds(t * TILE, TILE)
                xst = xs_ref[sl, :]
                idxt = idx_ref[sl, :]
                acc = out_ref[sl, :]
                for k in range(E_PER):
                    e = owner * E_PER + k
                    xm = jnp.where(idxt == e, xst, jnp.bfloat16(0.0))
                    acc = acc + jnp.dot(xm, comm_ref[slot, k],
                                        preferred_element_type=jnp.float32)
                out_ref[sl, :] = acc
                return 0

            lax.fori_loop(0, n_tiles, tile_body, 0)

            if h < N_DEV - 1:
                rdma.wait()

    return pl.pallas_call(
        body,
        out_shape=jax.ShapeDtypeStruct((m_per, d_ff), jnp.float32),
        in_specs=[pl.BlockSpec(memory_space=pltpu.VMEM)] * 5,
        out_specs=pl.BlockSpec(memory_space=pltpu.VMEM),
        scratch_shapes=[
            pltpu.VMEM((m_per, d_model), jnp.bfloat16),
            pltpu.VMEM((d_model, d_ff), jnp.bfloat16),
            pltpu.VMEM((2, e_per, d_model, d_ff), jnp.bfloat16),
            pltpu.SemaphoreType.DMA((2,)),
            pltpu.SemaphoreType.DMA((2,)),
        ],
        compiler_params=pltpu.CompilerParams(collective_id=0),
    )(x, router_W, route_idx, expert_W, shared_W)


# device time: 415465 ns/iter; 1.7632x vs baseline; 1.7632x over previous
import jax
import jax.numpy as jnp
from jax import lax
from jax.experimental import pallas as pl
from jax.experimental.pallas import tpu as pltpu

N_DEV = 16
HALF = N_DEV // 2
E_PER = 4
TILE = 256


def kernel(x, router_W, route_idx, expert_W, shared_W):
    m_per, d_model = x.shape
    e_per, _, d_ff = expert_W.shape
    n_exp = router_W.shape[1]
    n_tiles = m_per // TILE

    def body(x_ref, rw_ref, idx_ref, ew_ref, sw_ref, out_ref,
             xs_ref, swb_ref, comm_r, comm_l,
             send_r, recv_r, send_l, recv_l):
        my_pos = lax.axis_index("i")
        left = lax.rem(my_pos - 1 + N_DEV, N_DEV)
        right = lax.rem(my_pos + 1, N_DEV)

        barrier_sem = pltpu.get_barrier_semaphore()
        for nbr in [left, right]:
            pl.semaphore_signal(
                barrier_sem, inc=1,
                device_id=(nbr,), device_id_type=pl.DeviceIdType.MESH,
            )
        pl.semaphore_wait(barrier_sem, 2)

        for k in range(E_PER):
            blk = ew_ref[k].astype(jnp.bfloat16)
            comm_r[0, k] = blk
            comm_l[0, k] = blk
        swb_ref[...] = sw_ref[...].astype(jnp.bfloat16)

        def prep(t, _):
            sl = pl.ds(t * TILE, TILE)
            xt = x_ref[sl, :]
            scores = jnp.dot(xt, rw_ref[...],
                             preferred_element_type=jnp.float32)
            s_max = jnp.max(scores, axis=-1, keepdims=True)
            p = jnp.exp(scores - s_max)
            p = p / jnp.sum(p, axis=-1, keepdims=True)
            idxt = idx_ref[sl, :]
            cols = lax.broadcasted_iota(jnp.int32, (TILE, n_exp), 1)
            psel = jnp.sum(jnp.where(cols == idxt, p, 0.0), axis=-1,
                           keepdims=True)
            xs_ref[sl, :] = (xt * psel).astype(jnp.bfloat16)
            out_ref[sl, :] = jnp.dot(xt.astype(jnp.bfloat16), swb_ref[...],
                                     preferred_element_type=jnp.float32)
            return 0

        lax.fori_loop(0, n_tiles, prep, 0)

        def process_block(comm, slot, owner):
            def tile_body(t, _):
                sl = pl.ds(t * TILE, TILE)
                xst = xs_ref[sl, :]
                idxt = idx_ref[sl, :]
                acc = out_ref[sl, :]
                for k in range(E_PER):
                    e = owner * E_PER + k
                    xm = jnp.where(idxt == e, xst, jnp.bfloat16(0.0))
                    acc = acc + jnp.dot(xm, comm[slot, k],
                                        preferred_element_type=jnp.float32)
                out_ref[sl, :] = acc
                return 0

            lax.fori_loop(0, n_tiles, tile_body, 0)

        for h in range(HALF + 1):
            slot = h % 2
            nxt = (h + 1) % 2
            rdmas = []
            if h < HALF:
                rdma = pltpu.make_async_remote_copy(
                    src_ref=comm_r.at[slot],
                    dst_ref=comm_r.at[nxt],
                    send_sem=send_r.at[slot],
                    recv_sem=recv_r.at[nxt],
                    device_id=(right,),
                    device_id_type=pl.DeviceIdType.MESH,
                )
                rdma.start()
                rdmas.append(rdma)
            if h < HALF - 1:
                rdma = pltpu.make_async_remote_copy(
                    src_ref=comm_l.at[slot],
                    dst_ref=comm_l.at[nxt],
                    send_sem=send_l.at[slot],
                    recv_sem=recv_l.at[nxt],
                    device_id=(left,),
                    device_id_type=pl.DeviceIdType.MESH,
                )
                rdma.start()
                rdmas.append(rdma)

            if h == 0:
                process_block(comm_r, 0, my_pos)
            else:
                process_block(comm_r, slot,
                              lax.rem(my_pos - h + N_DEV, N_DEV))
                if h <= HALF - 1:
                    process_block(comm_l, slot,
                                  lax.rem(my_pos + h, N_DEV))

            for rdma in rdmas:
                rdma.wait()

    return pl.pallas_call(
        body,
        out_shape=jax.ShapeDtypeStruct((m_per, d_ff), jnp.float32),
        in_specs=[pl.BlockSpec(memory_space=pltpu.VMEM)] * 5,
        out_specs=pl.BlockSpec(memory_space=pltpu.VMEM),
        scratch_shapes=[
            pltpu.VMEM((m_per, d_model), jnp.bfloat16),
            pltpu.VMEM((d_model, d_ff), jnp.bfloat16),
            pltpu.VMEM((2, e_per, d_model, d_ff), jnp.bfloat16),
            pltpu.VMEM((2, e_per, d_model, d_ff), jnp.bfloat16),
            pltpu.SemaphoreType.DMA((2,)),
            pltpu.SemaphoreType.DMA((2,)),
            pltpu.SemaphoreType.DMA((2,)),
            pltpu.SemaphoreType.DMA((2,)),
        ],
        compiler_params=pltpu.CompilerParams(collective_id=0),
    )(x, router_W, route_idx, expert_W, shared_W)
